# Initial kernel scaffold; baseline (speedup 1.0000x reference)
import jax
import jax.numpy as jnp
from jax.experimental import pallas as pl


def kernel(qkvp, rpb_table, proj_w, proj_b, rpi):
    raise NotImplementedError("write your pallas kernel here")


# speedup vs baseline: 214.7485x; 214.7485x over previous
import jax
import jax.numpy as jnp
from jax.experimental import pallas as pl

DIM = 192
NUM_HEADS = 6
HEAD = 32
WS = 16
N = 256
TOPK = 128
SCALE = HEAD ** -0.5
NRPB = (2 * WS - 1) ** 2
NRPB_PAD = 1024
CHUNK = 2048
ONE_BITS = 0x3F800001


def _bias_kernel(rpi_ref, table_t_ref, out_ref):
    idx = rpi_ref[0]
    iota = jax.lax.broadcasted_iota(jnp.int32, (NRPB_PAD, CHUNK), 0)
    oh = (iota == idx).astype(jnp.float32)
    out_ref[...] = jax.lax.dot_general(
        table_t_ref[...], oh, (((1,), (0,)), ((), ())),
        preferred_element_type=jnp.float32,
        precision=jax.lax.Precision.HIGHEST)


def _attn_kernel(x_ref, bias_ref, pw_ref, pb_ref, out_ref):
    x = x_ref[0]
    outs = []
    for h in range(NUM_HEADS):
        q = x[:, h * HEAD:(h + 1) * HEAD] * SCALE
        k = x[:, DIM + h * HEAD:DIM + (h + 1) * HEAD]
        v = x[:, 2 * DIM + h * HEAD:2 * DIM + (h + 1) * HEAD]
        lepe = x[:, 3 * DIM + h * HEAD:3 * DIM + (h + 1) * HEAD]
        logits = jax.lax.dot_general(
            q, k, (((1,), (1,)), ((), ())),
            preferred_element_type=jnp.float32,
            precision=jax.lax.Precision.HIGHEST)
        logits = logits + bias_ref[h]
        m = jnp.max(logits, axis=1, keepdims=True)
        e = jnp.exp(logits - m)
        ki = jax.lax.bitcast_convert_type(e, jnp.int32)
        lo = jnp.zeros((N, 1), jnp.int32)
        hi = jnp.full((N, 1), ONE_BITS, jnp.int32)
        for _ in range(30):
            mid = jax.lax.shift_right_logical(lo + hi, 1)
            cnt = jnp.sum((ki >= mid).astype(jnp.float32), axis=1, keepdims=True)
            ok = cnt >= float(TOPK)
            lo = jnp.where(ok, mid, lo)
            hi = jnp.where(ok, hi, mid)
        pm = jnp.where(ki >= lo, e, 0.0)
        s = jnp.sum(e, axis=1, keepdims=True)
        oh_ = jax.lax.dot_general(
            pm, v, (((1,), (0,)), ((), ())),
            preferred_element_type=jnp.float32,
            precision=jax.lax.Precision.HIGHEST)
        outs.append(oh_ / s + lepe)
    y = jnp.concatenate(outs, axis=1)
    out_ref[0] = jax.lax.dot_general(
        y, pw_ref[...], (((1,), (1,)), ((), ())),
        preferred_element_type=jnp.float32,
        precision=jax.lax.Precision.HIGHEST) + pb_ref[...]


def kernel(qkvp, rpb_table, proj_w, proj_b, rpi):
    b_, n, c4 = qkvp.shape
    table_t = jnp.pad(rpb_table, ((0, NRPB_PAD - NRPB), (0, 0))).T
    rpi3 = rpi.reshape(N * N // CHUNK, 1, CHUNK)
    bias = pl.pallas_call(
        _bias_kernel,
        grid=(N * N // CHUNK,),
        in_specs=[
            pl.BlockSpec((1, 1, CHUNK), lambda i: (i, 0, 0)),
            pl.BlockSpec((NUM_HEADS, NRPB_PAD), lambda i: (0, 0)),
        ],
        out_specs=pl.BlockSpec((NUM_HEADS, CHUNK), lambda i: (0, i)),
        out_shape=jax.ShapeDtypeStruct((NUM_HEADS, N * N), jnp.float32),
    )(rpi3, table_t)
    bias = bias.reshape(NUM_HEADS, N, N)
    out = pl.pallas_call(
        _attn_kernel,
        grid=(b_,),
        in_specs=[
            pl.BlockSpec((1, N, 4 * DIM), lambda i: (i, 0, 0)),
            pl.BlockSpec((NUM_HEADS, N, N), lambda i: (0, 0, 0)),
            pl.BlockSpec((DIM, DIM), lambda i: (0, 0)),
            pl.BlockSpec((1, DIM), lambda i: (0, 0)),
        ],
        out_specs=pl.BlockSpec((1, N, DIM), lambda i: (i, 0, 0)),
        out_shape=jax.ShapeDtypeStruct((b_, N, DIM), jnp.float32),
    )(qkvp, bias, proj_w, proj_b.reshape(1, DIM))
    return out


# speedup vs baseline: 309.6937x; 1.4421x over previous
import jax
import jax.numpy as jnp
from jax.experimental import pallas as pl

DIM = 192
NUM_HEADS = 6
HEAD = 32
WS = 16
N = 256
TOPK = 128
SCALE = HEAD ** -0.5
NRPB = (2 * WS - 1) ** 2
NRPB_PAD = 1024
CHUNK = 2048
ONE_BITS = 0x3F800001
SEARCH_ITERS = 20


def _bias_kernel(rpi_ref, table_t_ref, out_ref):
    idx = rpi_ref[0]
    iota = jax.lax.broadcasted_iota(jnp.int32, (NRPB_PAD, CHUNK), 0)
    oh = (iota == idx).astype(jnp.float32)
    out_ref[...] = jax.lax.dot_general(
        table_t_ref[...], oh, (((1,), (0,)), ((), ())),
        preferred_element_type=jnp.float32,
        precision=jax.lax.Precision.HIGHEST)


def _attn_kernel(x_ref, bias_ref, pw_ref, pb_ref, out_ref):
    x = x_ref[0]
    outs = []
    for h in range(NUM_HEADS):
        q = (x[:, h * HEAD:(h + 1) * HEAD] * SCALE).astype(jnp.bfloat16)
        k = x[:, DIM + h * HEAD:DIM + (h + 1) * HEAD].astype(jnp.bfloat16)
        v = x[:, 2 * DIM + h * HEAD:2 * DIM + (h + 1) * HEAD].astype(jnp.bfloat16)
        lepe = x[:, 3 * DIM + h * HEAD:3 * DIM + (h + 1) * HEAD]
        logits = jax.lax.dot_general(
            q, k, (((1,), (1,)), ((), ())),
            preferred_element_type=jnp.float32)
        logits = logits + bias_ref[h]
        m = jnp.max(logits, axis=1, keepdims=True)
        e = jnp.exp(logits - m)
        ki = jax.lax.bitcast_convert_type(e, jnp.int32)
        lo = jnp.min(ki, axis=1, keepdims=True)
        hi = jnp.full((N, 1), ONE_BITS, jnp.int32)
        for _ in range(SEARCH_ITERS):
            mid = jax.lax.shift_right_logical(lo + hi, 1)
            cnt = jnp.sum((ki >= mid).astype(jnp.float32), axis=1, keepdims=True)
            ok = cnt >= float(TOPK)
            lo = jnp.where(ok, mid, lo)
            hi = jnp.where(ok, hi, mid)
        pm = jnp.where(ki >= lo, e, 0.0).astype(jnp.bfloat16)
        s = jnp.sum(e, axis=1, keepdims=True)
        oh_ = jax.lax.dot_general(
            pm, v, (((1,), (0,)), ((), ())),
            preferred_element_type=jnp.float32)
        outs.append(oh_ / s + lepe)
    y = jnp.concatenate(outs, axis=1)
    out_ref[0] = jax.lax.dot_general(
        y, pw_ref[...], (((1,), (1,)), ((), ())),
        preferred_element_type=jnp.float32,
        precision=jax.lax.Precision.HIGHEST) + pb_ref[...]


def kernel(qkvp, rpb_table, proj_w, proj_b, rpi):
    b_, n, c4 = qkvp.shape
    table_t = jnp.pad(rpb_table, ((0, NRPB_PAD - NRPB), (0, 0))).T
    rpi3 = rpi.reshape(N * N // CHUNK, 1, CHUNK)
    bias = pl.pallas_call(
        _bias_kernel,
        grid=(N * N // CHUNK,),
        in_specs=[
            pl.BlockSpec((1, 1, CHUNK), lambda i: (i, 0, 0)),
            pl.BlockSpec((NUM_HEADS, NRPB_PAD), lambda i: (0, 0)),
        ],
        out_specs=pl.BlockSpec((NUM_HEADS, CHUNK), lambda i: (0, i)),
        out_shape=jax.ShapeDtypeStruct((NUM_HEADS, N * N), jnp.float32),
    )(rpi3, table_t)
    bias = bias.reshape(NUM_HEADS, N, N)
    out = pl.pallas_call(
        _attn_kernel,
        grid=(b_,),
        in_specs=[
            pl.BlockSpec((1, N, 4 * DIM), lambda i: (i, 0, 0)),
            pl.BlockSpec((NUM_HEADS, N, N), lambda i: (0, 0, 0)),
            pl.BlockSpec((DIM, DIM), lambda i: (0, 0)),
            pl.BlockSpec((1, DIM), lambda i: (0, 0)),
        ],
        out_specs=pl.BlockSpec((1, N, DIM), lambda i: (i, 0, 0)),
        out_shape=jax.ShapeDtypeStruct((b_, N, DIM), jnp.float32),
    )(qkvp, bias, proj_w, proj_b.reshape(1, DIM))
    return out
